# baseline (speedup 1.0000x reference)
import functools

import jax
import jax.numpy as jnp
from jax import lax
from jax.experimental import pallas as pl
from jax.experimental.pallas import tpu as pltpu
from jax.experimental.pallas import tpu_sc as plsc

NC = 2
NS = 16
L = 16
NW = NC * NS


def _sc_call(pts_T, gi_w, si_w, wpack, qpack, wbf_t):
    B = pts_T.shape[0]
    C = 3
    N = pts_T.shape[1] // C
    PW = si_w.shape[1]
    S = gi_w.shape[1] // PW
    WPB = NW // B
    NBLK = PW // L
    NWV = wpack.shape[0] // L

    mesh = plsc.VectorSubcoreMesh(
        core_axis_name="c", subcore_axis_name="s",
        num_cores=NC, num_subcores=NS)

    @functools.partial(
        pl.kernel,
        out_type=jax.ShapeDtypeStruct((NW, C * PW), jnp.float32),
        mesh=mesh,
        scratch_types=[
            pltpu.VMEM((C * N,), jnp.float32),
            pltpu.VMEM((S * PW,), jnp.int32),
            pltpu.VMEM((PW,), jnp.int32),
            pltpu.VMEM((wpack.shape[0],), jnp.float32),
            pltpu.VMEM((B * L,), jnp.float32),
            pltpu.VMEM((C * PW,), jnp.float32),
            pltpu.VMEM((8 * (S // 2) * 8 * L,), jnp.int32),
            pltpu.VMEM((8 * (S // 2) * 8 * L,), jnp.int32),
            pltpu.VMEM((192 * L,), jnp.int32),
            pltpu.VMEM((8 * 3 * L,), jnp.float32),
            pltpu.VMEM((8 * 3 * L,), jnp.float32),
        ],
        compiler_params=pltpu.CompilerParams(needs_layout_passes=False),
    )
    def k(pts_hbm, gi_hbm, si_hbm, wpack_hbm, qpack_hbm, wbf_hbm, out_hbm,
          pts_v, gi_v, si_v, wpack_v, qpack_v, out_v, h1_v, h2_v, wbf_v,
          cbuf, fbuf):
        wid = lax.axis_index("s") * NC + lax.axis_index("c")
        b = wid // WPB
        pltpu.sync_copy(pts_hbm.at[b], pts_v)
        pltpu.sync_copy(gi_hbm.at[wid], gi_v)
        pltpu.sync_copy(si_hbm.at[wid], si_v)
        pltpu.sync_copy(wpack_hbm, wpack_v)
        pltpu.sync_copy(qpack_hbm, qpack_v)
        pltpu.sync_copy(wbf_hbm, wbf_v)

        def wbf(kk):
            return plsc.bitcast(wbf_v[pl.ds(kk * L, L)], jnp.bfloat16)

        wvec = [wpack_v[pl.ds(i * L, L)] for i in range(NWV)]

        def wsc(k):
            return wvec[k // L][k % L]

        w1 = [[wsc(i * 8 + j) for j in range(8)] for i in range(3)]
        w2 = [[wsc(24 + i * 8 + j) for j in range(8)] for i in range(8)]
        w3 = [[wsc(88 + i * 16 + j) for j in range(16)] for i in range(8)]
        w4 = [[wsc(216 + i * 3 + j) for j in range(3)] for i in range(16)]
        qrow = qpack_v[pl.ds(pl.multiple_of(b * L, L), L)]
        qa = [qrow[i] for i in range(4)]
        qb = [qrow[4 + i] for i in range(4)]
        tt = [qrow[8 + i] for i in range(3)]

        off_n = jnp.full((L,), N, jnp.int32)

        def coords(idx):
            idx_y = idx + off_n
            idx_z = idx_y + off_n
            x = plsc.load_gather(pts_v, [idx])
            y = plsc.load_gather(pts_v, [idx_y])
            z = plsc.load_gather(pts_v, [idx_z])
            return x, y, z

        zero_f = jnp.zeros((L,), jnp.float32)
        zero_b = jnp.zeros((2 * L,), jnp.bfloat16)

        def blk_body(blk, carry):
            pblk = pl.multiple_of(blk * L, L)
            cidx = si_v[pl.ds(pblk, L)]
            cx, cy, cz = coords(cidx)

            def a_body(sp, c2):
                s4 = sp * 4
                rel = []
                for u in range(4):
                    ni = gi_v[pl.ds(
                        pl.multiple_of((s4 + u) * PW, L) + pblk, L)]
                    gx, gy, gz = coords(ni)
                    rel.append((gx - cx, gy - cy, gz - cz))
                soff = pl.multiple_of(sp * (16 * L), 16 * L)
                for j in range(8):
                    wx, wy, wz = w1[0][j], w1[1][j], w1[2][j]
                    h = [jnp.maximum(rel[u][0] * wx + rel[u][1] * wy
                                     + rel[u][2] * wz, 0.0)
                         for u in range(4)]
                    h1_v[pl.ds(soff + j * L, L)] = plsc.bitcast(plsc.pack(
                        h[0], h[1], format=plsc.PackFormat.INTERLEAVED),
                        jnp.int32)
                    h1_v[pl.ds(soff + (8 + j) * L, L)] = plsc.bitcast(
                        plsc.pack(h[2], h[3],
                                  format=plsc.PackFormat.INTERLEAVED),
                        jnp.int32)
                return c2
            lax.fori_loop(0, S // 4, a_body, 0)

            for half in range(2):
                cols = [[wbf(i * 8 + half * 4 + j) for i in range(8)]
                        for j in range(4)]

                def b_body(p2, c2, cols=cols, half=half):
                    poff = pl.multiple_of(p2 * (8 * L), 8 * L)
                    h1 = [plsc.bitcast(h1_v[pl.ds(poff + i * L, L)],
                                       jnp.bfloat16) for i in range(8)]
                    for j in range(4):
                        v = h1[0] * cols[j][0]
                        for i in range(1, 8):
                            v = v + h1[i] * cols[j][i]
                        h2_v[pl.ds(poff + (half * 4 + j) * L, L)] = (
                            plsc.bitcast(jnp.maximum(v, zero_b), jnp.int32))
                    return c2
                lax.fori_loop(0, S // 2, b_body, 0)

            fx, fy, fz = zero_f, zero_f, zero_f
            for q in range(4):
                cols = [[wbf(64 + i * 16 + q * 4 + j) for i in range(8)]
                        for j in range(4)]

                def c_body(p2, acc4, cols=cols):
                    poff = pl.multiple_of(p2 * (8 * L), 8 * L)
                    h2 = [plsc.bitcast(h2_v[pl.ds(poff + i * L, L)],
                                       jnp.bfloat16) for i in range(8)]
                    out4 = []
                    for j in range(4):
                        v = h2[0] * cols[j][0]
                        for i in range(1, 8):
                            v = v + h2[i] * cols[j][i]
                        out4.append(jnp.maximum(acc4[j], v))
                    return tuple(out4)
                acc4 = lax.fori_loop(0, S // 2, c_body,
                                     (zero_b, zero_b, zero_b, zero_b))
                for j in range(4):
                    c = q * 4 + j
                    fa, fb = plsc.unpack(
                        acc4[j], format=plsc.PackFormat.INTERLEAVED,
                        preferred_element_type=jnp.float32)
                    feat = jnp.maximum(fa, fb)
                    fx = fx + feat * w4[c][0]
                    fy = fy + feat * w4[c][1]
                    fz = fz + feat * w4[c][2]

            r0 = -(qa[1] * cx + qa[2] * cy + qa[3] * cz)
            r1 = qa[0] * cx - qa[2] * cz - qa[3] * cy
            r2 = qa[0] * cy - qa[1] * cz - qa[3] * cx
            r3 = qa[0] * cz - qa[1] * cy - qa[2] * cx
            wx = r0 * qb[1] - r1 * qb[0] - r2 * qb[3] - r3 * qb[2]
            wy = r0 * qb[2] - r1 * qb[3] - r2 * qb[0] - r3 * qb[1]
            wz = r0 * qb[3] - r1 * qb[2] - r2 * qb[1] - r3 * qb[0]

            out_v[pl.ds(pblk, L)] = wx + tt[0] + fx
            out_v[pl.ds(pblk + PW, L)] = wy + tt[1] + fy
            out_v[pl.ds(pblk + 2 * PW, L)] = wz + tt[2] + fz
            return carry

        lax.fori_loop(0, NBLK, blk_body, 0)
        pltpu.sync_copy(out_v, out_hbm.at[wid])

    return k(pts_T, gi_w, si_w, wpack, qpack, wbf_t)


def kernel(points, q, t, sample_idx, group_idx, W1, W2, W3, W4):
    B, N, _ = points.shape
    P = sample_idx.shape[1]
    S = group_idx.shape[2]
    WPB = NW // B
    PW = P // WPB

    qf = jnp.reshape(q, (B, 4)).astype(jnp.float32)
    qn = qf / (jnp.sqrt(jnp.sum(qf * qf, axis=-1, keepdims=True) + 1e-10)
               + 1e-10)
    q2 = jnp.sum(qn * qn, axis=-1, keepdims=True) + 1e-10
    qinv = jnp.concatenate([qn[:, 0:1], -qn[:, 1:4]], axis=-1) / q2

    pts_T = jnp.transpose(points.astype(jnp.float32),
                          (0, 2, 1)).reshape(B, 3 * N)
    gi_w = (jnp.transpose(group_idx.astype(jnp.int32), (0, 2, 1))
            .reshape(B, S, WPB, PW)
            .transpose(0, 2, 1, 3)
            .reshape(NW, S * PW))
    si_w = sample_idx.astype(jnp.int32).reshape(NW, PW)

    wpack = jnp.concatenate([
        W1.astype(jnp.float32).ravel(), W2.astype(jnp.float32).ravel(),
        W3.astype(jnp.float32).ravel(), W4.astype(jnp.float32).ravel(),
        jnp.zeros((8,), jnp.float32)])
    qpack = jnp.concatenate([
        qn, qinv, t.astype(jnp.float32),
        jnp.zeros((B, L - 11), jnp.float32)], axis=1).ravel()
    wbf_t = jax.lax.bitcast_convert_type(
        jnp.broadcast_to(
            jnp.concatenate([W2.astype(jnp.bfloat16).ravel(),
                             W3.astype(jnp.bfloat16).ravel()])[:, None, None],
            (192, L, 2)),
        jnp.int32).ravel()

    out = _sc_call(pts_T, gi_w, si_w, wpack, qpack, wbf_t)
    return (out.reshape(B, WPB, 3, PW)
            .transpose(0, 1, 3, 2)
            .reshape(B, P, 3))


# speedup vs baseline: 1.1120x; 1.1120x over previous
import functools

import jax
import jax.numpy as jnp
from jax import lax
from jax.experimental import pallas as pl
from jax.experimental.pallas import tpu as pltpu
from jax.experimental.pallas import tpu_sc as plsc

NC = 2
NS = 16
L = 16
NW = NC * NS


def _sc_call(pts_T, gi_w, si_w, wpack, qpack, wbf_t):
    B = pts_T.shape[0]
    C = 3
    N = pts_T.shape[1] // C
    PW = si_w.shape[1]
    S = gi_w.shape[1] // PW
    WPB = NW // B
    NBLK = PW // L
    NWV = wpack.shape[0] // L

    mesh = plsc.VectorSubcoreMesh(
        core_axis_name="c", subcore_axis_name="s",
        num_cores=NC, num_subcores=NS)

    @functools.partial(
        pl.kernel,
        out_type=jax.ShapeDtypeStruct((NW, C * PW), jnp.float32),
        mesh=mesh,
        scratch_types=[
            pltpu.VMEM((C * N,), jnp.float32),
            pltpu.VMEM((S * PW,), jnp.int32),
            pltpu.VMEM((PW,), jnp.int32),
            pltpu.VMEM((wpack.shape[0],), jnp.float32),
            pltpu.VMEM((B * L,), jnp.float32),
            pltpu.VMEM((C * PW,), jnp.float32),
            pltpu.VMEM(((S // 2) * 8 * L,), jnp.int32),
            pltpu.VMEM(((S // 2) * 8 * L,), jnp.int32),
            pltpu.VMEM((192 * L,), jnp.int32),
        ],
        compiler_params=pltpu.CompilerParams(needs_layout_passes=False),
    )
    def k(pts_hbm, gi_hbm, si_hbm, wpack_hbm, qpack_hbm, wbf_hbm, out_hbm,
          pts_v, gi_v, si_v, wpack_v, qpack_v, out_v, h1_v, h2_v, wbf_v):
        wid = lax.axis_index("s") * NC + lax.axis_index("c")
        b = wid // WPB
        pltpu.sync_copy(pts_hbm.at[b], pts_v)
        pltpu.sync_copy(gi_hbm.at[wid], gi_v)
        pltpu.sync_copy(si_hbm.at[wid], si_v)
        pltpu.sync_copy(wpack_hbm, wpack_v)
        pltpu.sync_copy(qpack_hbm, qpack_v)
        pltpu.sync_copy(wbf_hbm, wbf_v)

        def wbf(kk):
            return plsc.bitcast(wbf_v[pl.ds(kk * L, L)], jnp.bfloat16)

        wvec = [wpack_v[pl.ds(i * L, L)] for i in range(NWV)]

        def wsc(k):
            return wvec[k // L][k % L]

        w1 = [[wsc(i * 8 + j) for j in range(8)] for i in range(3)]
        w2 = [[wsc(24 + i * 8 + j) for j in range(8)] for i in range(8)]
        w3 = [[wsc(88 + i * 16 + j) for j in range(16)] for i in range(8)]
        w4 = [[wsc(216 + i * 3 + j) for j in range(3)] for i in range(16)]
        qrow = qpack_v[pl.ds(pl.multiple_of(b * L, L), L)]
        qa = [qrow[i] for i in range(4)]
        qb = [qrow[4 + i] for i in range(4)]
        tt = [qrow[8 + i] for i in range(3)]

        off_n = jnp.full((L,), N, jnp.int32)

        def coords(idx):
            idx_y = idx + off_n
            idx_z = idx_y + off_n
            x = plsc.load_gather(pts_v, [idx])
            y = plsc.load_gather(pts_v, [idx_y])
            z = plsc.load_gather(pts_v, [idx_z])
            return x, y, z

        zero_f = jnp.zeros((L,), jnp.float32)
        zero_b = jnp.zeros((2 * L,), jnp.bfloat16)

        def blk_body(blk, carry):
            pblk = pl.multiple_of(blk * L, L)
            cidx = si_v[pl.ds(pblk, L)]
            cx, cy, cz = coords(cidx)

            def a_body(sp, c2):
                s4 = sp * 4
                rel = []
                for u in range(4):
                    ni = gi_v[pl.ds(
                        pl.multiple_of((s4 + u) * PW, L) + pblk, L)]
                    gx, gy, gz = coords(ni)
                    rel.append((gx - cx, gy - cy, gz - cz))
                soff = pl.multiple_of(sp * (16 * L), 16 * L)
                for j in range(8):
                    wx, wy, wz = w1[0][j], w1[1][j], w1[2][j]
                    h = [jnp.maximum(rel[u][0] * wx + rel[u][1] * wy
                                     + rel[u][2] * wz, 0.0)
                         for u in range(4)]
                    h1_v[pl.ds(soff + j * L, L)] = plsc.bitcast(plsc.pack(
                        h[0], h[1], format=plsc.PackFormat.INTERLEAVED),
                        jnp.int32)
                    h1_v[pl.ds(soff + (8 + j) * L, L)] = plsc.bitcast(
                        plsc.pack(h[2], h[3],
                                  format=plsc.PackFormat.INTERLEAVED),
                        jnp.int32)
                return c2
            lax.fori_loop(0, S // 4, a_body, 0)

            for half in range(2):
                cols = [[wbf(i * 8 + half * 4 + j) for i in range(8)]
                        for j in range(4)]

                def b_body(p2, c2, cols=cols, half=half):
                    poff = pl.multiple_of(p2 * (8 * L), 8 * L)
                    h1 = [plsc.bitcast(h1_v[pl.ds(poff + i * L, L)],
                                       jnp.bfloat16) for i in range(8)]
                    for j in range(4):
                        v = h1[0] * cols[j][0]
                        for i in range(1, 8):
                            v = v + h1[i] * cols[j][i]
                        h2_v[pl.ds(poff + (half * 4 + j) * L, L)] = (
                            plsc.bitcast(jnp.maximum(v, zero_b), jnp.int32))
                    return c2
                lax.fori_loop(0, S // 2, b_body, 0)

            fx, fy, fz = zero_f, zero_f, zero_f
            for q in range(4):
                cols = [[wbf(64 + i * 16 + q * 4 + j) for i in range(8)]
                        for j in range(4)]

                def c_body(p2, acc4, cols=cols):
                    poff = pl.multiple_of(p2 * (8 * L), 8 * L)
                    h2 = [plsc.bitcast(h2_v[pl.ds(poff + i * L, L)],
                                       jnp.bfloat16) for i in range(8)]
                    out4 = []
                    for j in range(4):
                        v = h2[0] * cols[j][0]
                        for i in range(1, 8):
                            v = v + h2[i] * cols[j][i]
                        out4.append(jnp.maximum(acc4[j], v))
                    return tuple(out4)
                acc4 = lax.fori_loop(0, S // 2, c_body,
                                     (zero_b, zero_b, zero_b, zero_b))
                for j in range(4):
                    c = q * 4 + j
                    fa, fb = plsc.unpack(
                        acc4[j], format=plsc.PackFormat.INTERLEAVED,
                        preferred_element_type=jnp.float32)
                    feat = jnp.maximum(fa, fb)
                    fx = fx + feat * w4[c][0]
                    fy = fy + feat * w4[c][1]
                    fz = fz + feat * w4[c][2]

            r0 = -(qa[1] * cx + qa[2] * cy + qa[3] * cz)
            r1 = qa[0] * cx - qa[2] * cz - qa[3] * cy
            r2 = qa[0] * cy - qa[1] * cz - qa[3] * cx
            r3 = qa[0] * cz - qa[1] * cy - qa[2] * cx
            wx = r0 * qb[1] - r1 * qb[0] - r2 * qb[3] - r3 * qb[2]
            wy = r0 * qb[2] - r1 * qb[3] - r2 * qb[0] - r3 * qb[1]
            wz = r0 * qb[3] - r1 * qb[2] - r2 * qb[1] - r3 * qb[0]

            out_v[pl.ds(pblk, L)] = wx + tt[0] + fx
            out_v[pl.ds(pblk + PW, L)] = wy + tt[1] + fy
            out_v[pl.ds(pblk + 2 * PW, L)] = wz + tt[2] + fz
            return carry

        lax.fori_loop(0, NBLK, blk_body, 0)
        pltpu.sync_copy(out_v, out_hbm.at[wid])

    return k(pts_T, gi_w, si_w, wpack, qpack, wbf_t)


def kernel(points, q, t, sample_idx, group_idx, W1, W2, W3, W4):
    B, N, _ = points.shape
    P = sample_idx.shape[1]
    S = group_idx.shape[2]
    WPB = NW // B
    PW = P // WPB

    qf = jnp.reshape(q, (B, 4)).astype(jnp.float32)
    qn = qf / (jnp.sqrt(jnp.sum(qf * qf, axis=-1, keepdims=True) + 1e-10)
               + 1e-10)
    q2 = jnp.sum(qn * qn, axis=-1, keepdims=True) + 1e-10
    qinv = jnp.concatenate([qn[:, 0:1], -qn[:, 1:4]], axis=-1) / q2

    pts_T = jnp.transpose(points.astype(jnp.float32),
                          (0, 2, 1)).reshape(B, 3 * N)
    gi_w = (jnp.transpose(group_idx.astype(jnp.int32), (0, 2, 1))
            .reshape(B, S, WPB, PW)
            .transpose(0, 2, 1, 3)
            .reshape(NW, S * PW))
    si_w = sample_idx.astype(jnp.int32).reshape(NW, PW)

    wpack = jnp.concatenate([
        W1.astype(jnp.float32).ravel(), W2.astype(jnp.float32).ravel(),
        W3.astype(jnp.float32).ravel(), W4.astype(jnp.float32).ravel(),
        jnp.zeros((8,), jnp.float32)])
    qpack = jnp.concatenate([
        qn, qinv, t.astype(jnp.float32),
        jnp.zeros((B, L - 11), jnp.float32)], axis=1).ravel()
    wbf_t = jax.lax.bitcast_convert_type(
        jnp.broadcast_to(
            jnp.concatenate([W2.astype(jnp.bfloat16).ravel(),
                             W3.astype(jnp.bfloat16).ravel()])[:, None, None],
            (192, L, 2)),
        jnp.int32).ravel()

    out = _sc_call(pts_T, gi_w, si_w, wpack, qpack, wbf_t)
    return (out.reshape(B, WPB, 3, PW)
            .transpose(0, 1, 3, 2)
            .reshape(B, P, 3))
